# Initial kernel scaffold; baseline (speedup 1.0000x reference)
import jax
import jax.numpy as jnp
from jax.experimental import pallas as pl


def kernel(input):
    raise NotImplementedError("write your pallas kernel here")


# speedup vs baseline: 416.4447x; 416.4447x over previous
import functools

import jax
import jax.numpy as jnp
from jax import lax
from jax.experimental import pallas as pl
from jax.experimental.pallas import tpu as pltpu
from jax.experimental.pallas import tpu_sc as plsc

_H = 2048
_W = 2048
_L = 16
_NC = 2
_NS = 16
_NW = _NC * _NS
_ROWS_PER_W = _H // _NW


def _sc_body(in_hbm, x_hbm, cons_hbm, b0, b1, b2, b3, xe, xo, ce, co):
    wid = lax.axis_index("s") * _NC + lax.axis_index("c")
    base = wid * _ROWS_PER_W

    iota = lax.broadcasted_iota(jnp.int32, (_L,), 0)
    idx_even0 = iota * 2
    ones = jnp.full((_L,), 1.0, dtype=jnp.float32)

    def init_body(k, carry):
        plsc.store_scatter(ce, [idx_even0 + 2 * k * _L], ones)
        return carry

    lax.fori_loop(0, _W // _L, init_body, 0)

    def row_body(r, carry):
        a = base + r
        pltpu.sync_copy(in_hbm.at[0, a], b0)
        pltpu.sync_copy(in_hbm.at[1, a], b1)
        pltpu.sync_copy(in_hbm.at[2, a], b2)
        pltpu.sync_copy(in_hbm.at[3, a], b3)

        def col_body(k, c2):
            off = k * _L
            a0 = b0[pl.ds(off, _L)]
            a1 = b1[pl.ds(off, _L)]
            a2 = b2[pl.ds(off, _L)]
            a3 = b3[pl.ds(off, _L)]
            ie = idx_even0 + 2 * off
            io = ie + 1
            plsc.store_scatter(xe, [ie], a0)
            plsc.store_scatter(xe, [io], a2)
            plsc.store_scatter(xo, [ie], a1)
            plsc.store_scatter(xo, [io], a3)
            t = a0 + a1 + 1.0
            s2 = (a0 + 1.0) * 0.5
            s1 = t * (1.0 / 3.0)
            s0 = (t + a2) * 0.25
            plsc.store_scatter(ce, [io], s1)
            plsc.store_scatter(co, [ie], s2)
            plsc.store_scatter(co, [io], s0)
            return c2

        lax.fori_loop(0, _W // _L, col_body, 0)

        pltpu.sync_copy(xe, x_hbm.at[2 * a])
        pltpu.sync_copy(xo, x_hbm.at[2 * a + 1])
        pltpu.sync_copy(ce, cons_hbm.at[2 * a])
        pltpu.sync_copy(co, cons_hbm.at[2 * a + 1])
        return carry

    lax.fori_loop(0, _ROWS_PER_W, row_body, 0)


@jax.jit
def _unsqueeze_cons(inp):
    mesh = plsc.VectorSubcoreMesh(core_axis_name="c", subcore_axis_name="s")
    run = functools.partial(
        pl.kernel,
        out_type=[
            jax.ShapeDtypeStruct((2 * _H, 2 * _W), jnp.float32),
            jax.ShapeDtypeStruct((2 * _H, 2 * _W), jnp.float32),
        ],
        mesh=mesh,
        compiler_params=pltpu.CompilerParams(needs_layout_passes=False),
        scratch_types=[
            pltpu.VMEM((_W,), jnp.float32),
            pltpu.VMEM((_W,), jnp.float32),
            pltpu.VMEM((_W,), jnp.float32),
            pltpu.VMEM((_W,), jnp.float32),
            pltpu.VMEM((2 * _W,), jnp.float32),
            pltpu.VMEM((2 * _W,), jnp.float32),
            pltpu.VMEM((2 * _W,), jnp.float32),
            pltpu.VMEM((2 * _W,), jnp.float32),
        ],
    )(_sc_body)
    return run(inp)


def kernel(input):
    B, C, H, W = input.shape
    assert (B, C, H, W) == (1, 4, _H, _W)
    x, cons = _unsqueeze_cons(input[0])
    x = x.reshape(1, 1, 2 * _H, 2 * _W)
    cons = cons.reshape(1, 1, 2 * _H, 2 * _W)
    return (x, cons)


# speedup vs baseline: 460.1083x; 1.1048x over previous
import functools

import jax
import jax.numpy as jnp
from jax import lax
from jax.experimental import pallas as pl
from jax.experimental.pallas import tpu as pltpu
from jax.experimental.pallas import tpu_sc as plsc

_H = 2048
_W = 2048
_L = 16
_NC = 2
_NS = 16
_NW = _NC * _NS
_ROWS_PER_W = _H // _NW
_R = 2
_NB = _ROWS_PER_W // _R
_NT = _NB // 2
_W2 = 2 * _W


def _sc_body(in_hbm, x_hbm, cons_hbm,
             inb0, inb1, xb0, xb1, cb0, cb1, si0, si1, so0, so1):
    wid = lax.axis_index("s") * _NC + lax.axis_index("c")
    base = wid * _ROWS_PER_W
    inb = (inb0, inb1)
    xb = (xb0, xb1)
    cb = (cb0, cb1)
    si = (si0, si1)
    so = (so0, so1)

    iota = lax.broadcasted_iota(jnp.int32, (_L,), 0)
    i2 = iota * 2
    ones = jnp.full((_L,), 1.0, dtype=jnp.float32)

    for s in range(2):
        def init_body(k, c, _cb=cb[s]):
            ie = i2 + 2 * k * _L
            plsc.store_scatter(_cb, [ie], ones)
            plsc.store_scatter(_cb, [ie + 2 * _W2], ones)
            return c

        lax.fori_loop(0, _W // _L, init_body, 0)

    def in_cps(g, s):
        a = base + _R * g
        return [
            pltpu.make_async_copy(
                in_hbm.at[pl.ds(c * _H * _W + a * _W, _R * _W)],
                inb[s].at[pl.ds(c * _R * _W, _R * _W)],
                si[s])
            for c in range(4)
        ]

    def out_cps(g, s):
        a = base + _R * g
        off = (2 * a) * _W2
        n = 2 * _R * _W2
        return [
            pltpu.make_async_copy(xb[s], x_hbm.at[pl.ds(off, n)], so[s]),
            pltpu.make_async_copy(cb[s], cons_hbm.at[pl.ds(off, n)], so[s]),
        ]

    def compute(s):
        _inb, _xb, _cb = inb[s], xb[s], cb[s]
        for r in range(_R):
            ib = r * _W
            xe_o = (2 * r) * _W2
            xo_o = (2 * r + 1) * _W2

            def col_body(k, _c):
                off = k * _L
                a0 = _inb[pl.ds(0 * _R * _W + ib + off, _L)]
                a1 = _inb[pl.ds(1 * _R * _W + ib + off, _L)]
                a2 = _inb[pl.ds(2 * _R * _W + ib + off, _L)]
                a3 = _inb[pl.ds(3 * _R * _W + ib + off, _L)]
                ie = i2 + 2 * off
                plsc.store_scatter(_xb, [ie + xe_o], a0)
                plsc.store_scatter(_xb, [ie + (xe_o + 1)], a2)
                plsc.store_scatter(_xb, [ie + xo_o], a1)
                plsc.store_scatter(_xb, [ie + (xo_o + 1)], a3)
                t = a0 + a1 + 1.0
                v2 = (a0 + 1.0) * 0.5
                v1 = t * (1.0 / 3.0)
                v0 = (t + a2) * 0.25
                plsc.store_scatter(_cb, [ie + (xe_o + 1)], v1)
                plsc.store_scatter(_cb, [ie + xo_o], v2)
                plsc.store_scatter(_cb, [ie + (xo_o + 1)], v0)
                return _c

            lax.fori_loop(0, _W // _L, col_body, 0)

    for c in in_cps(0, 0):
        c.start()
    for c in in_cps(1, 1):
        c.start()

    def run_block(t, s):
        g = 2 * t + s
        for c in in_cps(g, s):
            c.wait()

        @pl.when(t >= 1)
        def _():
            for c in out_cps(g - 2, s):
                c.wait()

        compute(s)
        for c in out_cps(g, s):
            c.start()

        @pl.when(t < _NT - 1)
        def _():
            for c in in_cps(g + 2, s):
                c.start()

    def main_body(t, carry):
        run_block(t, 0)
        run_block(t, 1)
        return carry

    lax.fori_loop(0, _NT, main_body, 0)

    for c in out_cps(2 * (_NT - 1), 0):
        c.wait()
    for c in out_cps(2 * (_NT - 1) + 1, 1):
        c.wait()


@jax.jit
def _unsqueeze_cons(inp_flat):
    mesh = plsc.VectorSubcoreMesh(core_axis_name="c", subcore_axis_name="s")
    run = functools.partial(
        pl.kernel,
        out_type=[
            jax.ShapeDtypeStruct((2 * _H * 2 * _W,), jnp.float32),
            jax.ShapeDtypeStruct((2 * _H * 2 * _W,), jnp.float32),
        ],
        mesh=mesh,
        compiler_params=pltpu.CompilerParams(needs_layout_passes=False),
        scratch_types=[
            pltpu.VMEM((4 * _R * _W,), jnp.float32),
            pltpu.VMEM((4 * _R * _W,), jnp.float32),
            pltpu.VMEM((2 * _R * _W2,), jnp.float32),
            pltpu.VMEM((2 * _R * _W2,), jnp.float32),
            pltpu.VMEM((2 * _R * _W2,), jnp.float32),
            pltpu.VMEM((2 * _R * _W2,), jnp.float32),
            pltpu.SemaphoreType.DMA,
            pltpu.SemaphoreType.DMA,
            pltpu.SemaphoreType.DMA,
            pltpu.SemaphoreType.DMA,
        ],
    )(_sc_body)
    return run(inp_flat)


def kernel(input):
    B, C, H, W = input.shape
    assert (B, C, H, W) == (1, 4, _H, _W)
    x, cons = _unsqueeze_cons(input.reshape(-1))
    x = x.reshape(1, 1, 2 * _H, 2 * _W)
    cons = cons.reshape(1, 1, 2 * _H, 2 * _W)
    return (x, cons)


# speedup vs baseline: 462.7603x; 1.0058x over previous
import functools

import jax
import jax.numpy as jnp
from jax import lax
from jax.experimental import pallas as pl
from jax.experimental.pallas import tpu as pltpu
from jax.experimental.pallas import tpu_sc as plsc

_H = 2048
_W = 2048
_L = 16
_NC = 2
_NS = 16
_NW = _NC * _NS
_ROWS_PER_W = _H // _NW
_R = 2
_NB = _ROWS_PER_W // _R
_NT = _NB // 2
_W2 = 2 * _W
_U = 4


def _sc_body(in_hbm, x_hbm, cons_hbm,
             inb0, inb1, xb0, xb1, cb0, cb1, si0, si1, so0, so1):
    wid = lax.axis_index("s") * _NC + lax.axis_index("c")
    base = wid * _ROWS_PER_W
    inb = (inb0, inb1)
    xb = (xb0, xb1)
    cb = (cb0, cb1)
    si = (si0, si1)
    so = (so0, so1)

    iota = lax.broadcasted_iota(jnp.int32, (_L,), 0)
    i2 = iota * 2
    ones = jnp.full((_L,), 1.0, dtype=jnp.float32)

    for s in range(2):
        def init_body(k, c, _cb=cb[s]):
            ie = i2 + 2 * k * _L
            plsc.store_scatter(_cb, [ie], ones)
            plsc.store_scatter(_cb, [ie + 2 * _W2], ones)
            return c

        lax.fori_loop(0, _W // _L, init_body, 0)

    def in_cps(g, s):
        a = base + _R * g
        return [
            pltpu.make_async_copy(
                in_hbm.at[pl.ds(c * _H * _W + a * _W, _R * _W)],
                inb[s].at[pl.ds(c * _R * _W, _R * _W)],
                si[s])
            for c in range(4)
        ]

    def out_cps(g, s):
        a = base + _R * g
        off = (2 * a) * _W2
        n = 2 * _R * _W2
        return [
            pltpu.make_async_copy(xb[s], x_hbm.at[pl.ds(off, n)], so[s]),
            pltpu.make_async_copy(cb[s], cons_hbm.at[pl.ds(off, n)], so[s]),
        ]

    def compute(s):
        _inb, _xb, _cb = inb[s], xb[s], cb[s]
        for r in range(_R):
            ib = r * _W
            xe_o = (2 * r) * _W2
            xo_o = (2 * r + 1) * _W2

            def col_body(k, _c):
                for u in range(_U):
                    off = k * (_U * _L) + u * _L
                    a0 = _inb[pl.ds(0 * _R * _W + ib + off, _L)]
                    a1 = _inb[pl.ds(1 * _R * _W + ib + off, _L)]
                    a2 = _inb[pl.ds(2 * _R * _W + ib + off, _L)]
                    a3 = _inb[pl.ds(3 * _R * _W + ib + off, _L)]
                    ie = i2 + 2 * off
                    plsc.store_scatter(_xb, [ie + xe_o], a0)
                    plsc.store_scatter(_xb, [ie + (xe_o + 1)], a2)
                    plsc.store_scatter(_xb, [ie + xo_o], a1)
                    plsc.store_scatter(_xb, [ie + (xo_o + 1)], a3)
                    t = a0 + a1 + 1.0
                    v2 = (a0 + 1.0) * 0.5
                    v1 = t * (1.0 / 3.0)
                    v0 = (t + a2) * 0.25
                    plsc.store_scatter(_cb, [ie + (xe_o + 1)], v1)
                    plsc.store_scatter(_cb, [ie + xo_o], v2)
                    plsc.store_scatter(_cb, [ie + (xo_o + 1)], v0)
                return _c

            lax.fori_loop(0, _W // (_U * _L), col_body, 0)

    for c in in_cps(0, 0):
        c.start()
    for c in in_cps(1, 1):
        c.start()

    def run_block(t, s):
        g = 2 * t + s
        for c in in_cps(g, s):
            c.wait()

        @pl.when(t >= 1)
        def _():
            for c in out_cps(g - 2, s):
                c.wait()

        compute(s)
        for c in out_cps(g, s):
            c.start()

        @pl.when(t < _NT - 1)
        def _():
            for c in in_cps(g + 2, s):
                c.start()

    def main_body(t, carry):
        run_block(t, 0)
        run_block(t, 1)
        return carry

    lax.fori_loop(0, _NT, main_body, 0)

    for c in out_cps(2 * (_NT - 1), 0):
        c.wait()
    for c in out_cps(2 * (_NT - 1) + 1, 1):
        c.wait()


@jax.jit
def _unsqueeze_cons(inp_flat):
    mesh = plsc.VectorSubcoreMesh(core_axis_name="c", subcore_axis_name="s")
    run = functools.partial(
        pl.kernel,
        out_type=[
            jax.ShapeDtypeStruct((2 * _H * 2 * _W,), jnp.float32),
            jax.ShapeDtypeStruct((2 * _H * 2 * _W,), jnp.float32),
        ],
        mesh=mesh,
        compiler_params=pltpu.CompilerParams(needs_layout_passes=False),
        scratch_types=[
            pltpu.VMEM((4 * _R * _W,), jnp.float32),
            pltpu.VMEM((4 * _R * _W,), jnp.float32),
            pltpu.VMEM((2 * _R * _W2,), jnp.float32),
            pltpu.VMEM((2 * _R * _W2,), jnp.float32),
            pltpu.VMEM((2 * _R * _W2,), jnp.float32),
            pltpu.VMEM((2 * _R * _W2,), jnp.float32),
            pltpu.SemaphoreType.DMA,
            pltpu.SemaphoreType.DMA,
            pltpu.SemaphoreType.DMA,
            pltpu.SemaphoreType.DMA,
        ],
    )(_sc_body)
    return run(inp_flat)


def kernel(input):
    B, C, H, W = input.shape
    assert (B, C, H, W) == (1, 4, _H, _W)
    x, cons = _unsqueeze_cons(input.reshape(-1))
    x = x.reshape(1, 1, 2 * _H, 2 * _W)
    cons = cons.reshape(1, 1, 2 * _H, 2 * _W)
    return (x, cons)


# speedup vs baseline: 1338.8414x; 2.8932x over previous
import functools

import jax
import jax.numpy as jnp
from jax import lax
from jax.experimental import pallas as pl
from jax.experimental.pallas import tpu as pltpu
from jax.experimental.pallas import tpu_sc as plsc

_H = 2048
_W = 2048
_L = 16
_NC = 2
_NS = 16
_NW = _NC * _NS
_TR = _H // 8
_TC = _W // 128
_CPU = 4
_NCU = _TC // _CPU
_APW = _TR // _NW
_NU = _APW * _NCU
_NT = _NU // 2
_CH_STRIDE = _H * _W
_IN_SLAB = _CPU * 8 * 128
_OUT_SLAB = 2 * _CPU * 8 * 128
_OTR_STRIDE = (2 * _W // 128) * 8 * 128


def _sc_body(in_hbm, x_hbm, cons_hbm,
             inb0, inb1, xb00, xb01, xb10, xb11, cb00, cb01, cb10, cb11,
             si0, si1, so0, so1):
    wid = lax.axis_index("s") * _NC + lax.axis_index("c")
    a_base = wid * _APW
    inb = (inb0, inb1)
    xb = ((xb00, xb01), (xb10, xb11))
    cb = ((cb00, cb01), (cb10, cb11))
    si = (si0, si1)
    so = (so0, so1)

    iota = lax.broadcasted_iota(jnp.int32, (_L,), 0)
    i2 = iota * 2
    ones = jnp.full((_L,), 1.0, dtype=jnp.float32)

    for s in range(2):
        for u in range(2):
            def init_body(j, c, _cbu=cb[s][u]):
                tc = j // 16
                rr = 2 * ((j // 4) % 4)
                ck = j % 4
                base = tc * 1024 + rr * 128 + ck * 32
                plsc.store_scatter(_cbu, [i2 + base], ones)
                return c

            lax.fori_loop(0, 128, init_body, 0)

    def unit_coords(g):
        return a_base + g // _NCU, g % _NCU

    def in_cps(g, s):
        A, C = unit_coords(g)
        off = A * (_TC * 1024) + C * _IN_SLAB
        return [
            pltpu.make_async_copy(
                in_hbm.at[pl.ds(c * _CH_STRIDE + off, _IN_SLAB)],
                inb[s].at[pl.ds(c * _IN_SLAB, _IN_SLAB)],
                si[s])
            for c in range(4)
        ]

    def out_cps(g, s):
        A, C = unit_coords(g)
        cps = []
        for u in range(2):
            off = (2 * A + u) * _OTR_STRIDE + C * _OUT_SLAB
            cps.append(pltpu.make_async_copy(
                xb[s][u], x_hbm.at[pl.ds(off, _OUT_SLAB)], so[s]))
            cps.append(pltpu.make_async_copy(
                cb[s][u], cons_hbm.at[pl.ds(off, _OUT_SLAB)], so[s]))
        return cps

    def compute(s):
        _inb = inb[s]

        def col_body(k, _c):
            t_in = k // 8
            l_in = (k % 8) * _L
            in_off = t_in * 1024 + l_in
            tc_out = k // 4
            l_out = (k * 32) % 128
            out_off = tc_out * 1024 + l_out
            for ri in range(8):
                a0 = _inb[pl.ds(0 * _IN_SLAB + in_off + ri * 128, _L)]
                a1 = _inb[pl.ds(1 * _IN_SLAB + in_off + ri * 128, _L)]
                a2 = _inb[pl.ds(2 * _IN_SLAB + in_off + ri * 128, _L)]
                a3 = _inb[pl.ds(3 * _IN_SLAB + in_off + ri * 128, _L)]
                u0, rr0 = divmod(2 * ri, 8)
                u1, rr1 = divmod(2 * ri + 1, 8)
                ie0 = i2 + (out_off + rr0 * 128)
                io0 = ie0 + 1
                ie1 = i2 + (out_off + rr1 * 128)
                io1 = ie1 + 1
                plsc.store_scatter(xb[s][u0], [ie0], a0)
                plsc.store_scatter(xb[s][u0], [io0], a2)
                plsc.store_scatter(xb[s][u1], [ie1], a1)
                plsc.store_scatter(xb[s][u1], [io1], a3)
                t = a0 + a1 + 1.0
                v2 = (a0 + 1.0) * 0.5
                v1 = t * (1.0 / 3.0)
                v0 = (t + a2) * 0.25
                plsc.store_scatter(cb[s][u0], [io0], v1)
                plsc.store_scatter(cb[s][u1], [ie1], v2)
                plsc.store_scatter(cb[s][u1], [io1], v0)
            return _c

        lax.fori_loop(0, _CPU * 8, col_body, 0)

    for c in in_cps(0, 0):
        c.start()
    for c in in_cps(1, 1):
        c.start()

    def run_unit(t, s):
        g = 2 * t + s
        for c in in_cps(g, s):
            c.wait()

        @pl.when(t >= 1)
        def _():
            for c in out_cps(g - 2, s):
                c.wait()

        compute(s)
        for c in out_cps(g, s):
            c.start()

        @pl.when(t < _NT - 1)
        def _():
            for c in in_cps(g + 2, s):
                c.start()

    def main_body(t, carry):
        run_unit(t, 0)
        run_unit(t, 1)
        return carry

    lax.fori_loop(0, _NT, main_body, 0)

    for c in out_cps(2 * (_NT - 1), 0):
        c.wait()
    for c in out_cps(2 * (_NT - 1) + 1, 1):
        c.wait()


@jax.jit
def _unsqueeze_cons(inp_flat):
    mesh = plsc.VectorSubcoreMesh(core_axis_name="c", subcore_axis_name="s")
    run = functools.partial(
        pl.kernel,
        out_type=[
            jax.ShapeDtypeStruct((4 * _H * _W,), jnp.float32),
            jax.ShapeDtypeStruct((4 * _H * _W,), jnp.float32),
        ],
        mesh=mesh,
        compiler_params=pltpu.CompilerParams(needs_layout_passes=False),
        scratch_types=[
            pltpu.VMEM((4 * _IN_SLAB,), jnp.float32),
            pltpu.VMEM((4 * _IN_SLAB,), jnp.float32),
            pltpu.VMEM((_OUT_SLAB,), jnp.float32),
            pltpu.VMEM((_OUT_SLAB,), jnp.float32),
            pltpu.VMEM((_OUT_SLAB,), jnp.float32),
            pltpu.VMEM((_OUT_SLAB,), jnp.float32),
            pltpu.VMEM((_OUT_SLAB,), jnp.float32),
            pltpu.VMEM((_OUT_SLAB,), jnp.float32),
            pltpu.VMEM((_OUT_SLAB,), jnp.float32),
            pltpu.VMEM((_OUT_SLAB,), jnp.float32),
            pltpu.SemaphoreType.DMA,
            pltpu.SemaphoreType.DMA,
            pltpu.SemaphoreType.DMA,
            pltpu.SemaphoreType.DMA,
        ],
    )(_sc_body)
    return run(inp_flat)


def kernel(input):
    B, C, H, W = input.shape
    assert (B, C, H, W) == (1, 4, _H, _W)
    inp_flat = (input.reshape(4, _TR, 8, _TC, 128)
                .transpose(0, 1, 3, 2, 4)
                .reshape(-1))
    x_flat, cons_flat = _unsqueeze_cons(inp_flat)
    out_tr = 2 * _H // 8
    out_tc = 2 * _W // 128

    def detile(f):
        return (f.reshape(out_tr, out_tc, 8, 128)
                .transpose(0, 2, 1, 3)
                .reshape(1, 1, 2 * _H, 2 * _W))

    return (detile(x_flat), detile(cons_flat))
